# Initial kernel scaffold; baseline (speedup 1.0000x reference)
import functools

import jax
import jax.numpy as jnp
from jax import lax
from jax.experimental import pallas as pl
from jax.experimental.pallas import tpu as pltpu
from jax.experimental.pallas import tpu_sc as plsc

NC = 2
NS = 16
NW = NC * NS
LANES = 16
K = 128


def _mm_body(x_ref, w_ref, d_ref, y_ref):
    y = jnp.dot(x_ref[...], w_ref[...], preferred_element_type=jnp.float32)
    y_ref[...] = y * d_ref[...]


def _matmul_scale(X, W, deg):
    n, d_in = X.shape
    d_out = W.shape[1]
    bm = 1000
    grid = (n // bm,)
    return pl.pallas_call(
        _mm_body,
        grid=grid,
        in_specs=[
            pl.BlockSpec((bm, d_in), lambda i: (i, 0)),
            pl.BlockSpec((d_in, d_out), lambda i: (0, 0)),
            pl.BlockSpec((bm, 1), lambda i: (i, 0)),
        ],
        out_specs=pl.BlockSpec((bm, d_out), lambda i: (i, 0)),
        out_shape=jax.ShapeDtypeStruct((n, d_out), jnp.float32),
    )(X, W, deg[:, None])


def _sc_aggregate(Y, rp_pad, col_pad, deg_pad, rpw):
    n_pad = NW * rpw
    d = Y.shape[1]
    ng = d // LANES

    mesh = plsc.VectorSubcoreMesh(core_axis_name="c", subcore_axis_name="s")

    @functools.partial(
        pl.kernel,
        out_type=jax.ShapeDtypeStruct((n_pad, d), jnp.float32),
        mesh=mesh,
        scratch_types=[
            pltpu.VMEM((rpw + 8,), jnp.int32),
            pltpu.VMEM((rpw,), jnp.float32),
            pltpu.VMEM((K,), jnp.int32),
            pltpu.VMEM((K, d), jnp.float32),
            pltpu.VMEM((d,), jnp.float32),
            pltpu.VMEM((rpw, d), jnp.float32),
            pltpu.SemaphoreType.DMA,
        ],
    )
    def agg(y_hbm, rp_hbm, colx_hbm, deg_hbm, out_hbm,
            rp_v, deg_v, col_v, rows_v, acc_v, out_v, sem):
        wid = lax.axis_index("s") * NC + lax.axis_index("c")
        r0 = wid * rpw
        pltpu.sync_copy(rp_hbm.at[pl.ds(r0, rpw + 8)], rp_v)
        pltpu.sync_copy(deg_hbm.at[pl.ds(r0, rpw)], deg_v)

        for g in range(ng):
            acc_v[pl.ds(g * LANES, LANES)] = jnp.zeros((LANES,), jnp.float32)

        e0 = rp_v[0]
        e1 = rp_v[rpw]
        base0 = (e0 // 8) * 8
        nchunks = (e1 - base0 + (K - 1)) // K

        def flush(cur):
            dscale = deg_v[cur]
            for g in range(ng):
                sl = pl.ds(g * LANES, LANES)
                out_v[cur, sl] = acc_v[sl] * dscale
                acc_v[sl] = jnp.zeros((LANES,), jnp.float32)

        def chunk_body(t, carry):
            cur, nb = carry
            base = base0 + t * K
            pltpu.sync_copy(colx_hbm.at[pl.ds(base, K)], col_v)
            pltpu.async_copy(y_hbm.at[col_v], rows_v, sem).wait()
            lo = jnp.maximum(e0, base)
            hi = jnp.minimum(e1, base + K)

            def edge_body(j, c):
                cur2, nb2 = c

                def wcond(c2):
                    return c2[1] <= j

                def wbody(c2):
                    cc, _ = c2
                    flush(cc)
                    return (cc + 1, rp_v[cc + 2])

                cur2, nb2 = lax.while_loop(wcond, wbody, (cur2, nb2))
                jj = j - base
                for g in range(ng):
                    sl = pl.ds(g * LANES, LANES)
                    plsc.addupdate(acc_v.at[sl], rows_v[jj, sl])
                return (cur2, nb2)

            return lax.fori_loop(lo, hi, edge_body, (cur, nb))

        cur, _ = lax.fori_loop(0, nchunks, chunk_body, (0, rp_v[1]))

        def tail_body(i, c):
            flush(i)
            return c

        lax.fori_loop(cur, rpw, tail_body, 0)

        pltpu.sync_copy(out_v, out_hbm.at[pl.ds(r0, rpw)])

    return agg(Y, rp_pad, col_pad, deg_pad)


def kernel(X, weights, row_pointers, column_index, degrees, partPtr, part2Node,
           threadPerBlock):
    n = X.shape[0]
    e = column_index.shape[0]

    rpw = (-(-n // NW) + 7) // 8 * 8
    n_pad = NW * rpw

    Y = _matmul_scale(X, weights, degrees)

    rp_pad = jnp.concatenate(
        [row_pointers,
         jnp.full((n_pad + 8 - (n + 1),), e, dtype=jnp.int32)])
    col_pad = jnp.concatenate(
        [column_index, jnp.zeros((K,), dtype=jnp.int32)])
    deg_pad = jnp.concatenate(
        [degrees, jnp.zeros((n_pad - n,), dtype=jnp.float32)])

    out = _sc_aggregate(Y, rp_pad, col_pad, deg_pad, rpw)
    return out[:n]


# speedup vs baseline: 9.2033x; 9.2033x over previous
import functools

import jax
import jax.numpy as jnp
from jax import lax
from jax.experimental import pallas as pl
from jax.experimental.pallas import tpu as pltpu
from jax.experimental.pallas import tpu_sc as plsc

NC = 2
NS = 16
NW = NC * NS
LANES = 16
K = 128


def _mm_body(x_ref, w_ref, d_ref, y_ref):
    y = jnp.dot(x_ref[...], w_ref[...], preferred_element_type=jnp.float32)
    y_ref[...] = y * d_ref[...]


def _matmul_scale(X, W, deg):
    n, d_in = X.shape
    d_out = W.shape[1]
    bm = 1000
    grid = (n // bm,)
    return pl.pallas_call(
        _mm_body,
        grid=grid,
        in_specs=[
            pl.BlockSpec((bm, d_in), lambda i: (i, 0)),
            pl.BlockSpec((d_in, d_out), lambda i: (0, 0)),
            pl.BlockSpec((bm, 1), lambda i: (i, 0)),
        ],
        out_specs=pl.BlockSpec((bm, d_out), lambda i: (i, 0)),
        out_shape=jax.ShapeDtypeStruct((n, d_out), jnp.float32),
    )(X, W, deg[:, None])


def _sc_aggregate(Y, rp_pad, col_pad, deg_pad, fr_pad, rpw):
    n_pad = NW * rpw
    d = Y.shape[1]
    ng = d // LANES
    nfr = fr_pad.shape[0]

    mesh = plsc.VectorSubcoreMesh(core_axis_name="c", subcore_axis_name="s")

    @functools.partial(
        pl.kernel,
        out_type=jax.ShapeDtypeStruct((n_pad, d), jnp.float32),
        mesh=mesh,
        scratch_types=[
            pltpu.VMEM((rpw + 24,), jnp.int32),
            pltpu.VMEM((rpw + 16,), jnp.float32),
            pltpu.VMEM((nfr,), jnp.int32),
            pltpu.VMEM((K,), jnp.int32),
            pltpu.VMEM((K, d), jnp.float32),
            pltpu.VMEM((rpw, d), jnp.float32),
            pltpu.SemaphoreType.DMA,
        ],
    )
    def agg(y_hbm, rp_hbm, colx_hbm, deg_hbm, fr_hbm, out_hbm,
            rp_v, deg_v, fr_v, col_v, rows_v, out_v, sem):
        wid = lax.axis_index("s") * NC + lax.axis_index("c")
        r0 = wid * rpw
        pltpu.sync_copy(rp_hbm.at[pl.ds(r0, rpw + 24)], rp_v)
        pltpu.sync_copy(deg_hbm.at[pl.ds(r0, rpw + 16)], deg_v)
        pltpu.sync_copy(fr_hbm, fr_v)

        def sread(ref, i):
            return ref[pl.ds(i, LANES)][0]

        zeros = jnp.zeros((LANES,), jnp.float32)

        def zero_body(r, c):
            for g in range(ng):
                out_v[r, pl.ds(g * LANES, LANES)] = zeros
            return c

        lax.fori_loop(0, rpw, zero_body, 0)

        e0 = sread(rp_v, 0)
        e1 = sread(rp_v, rpw)
        t0 = e0 // K
        t1 = (e1 + (K - 1)) // K

        def chunk_body(t, c):
            base = t * K
            pltpu.sync_copy(colx_hbm.at[pl.ds(base, K)], col_v)
            pltpu.async_copy(y_hbm.at[col_v], rows_v, sem).wait()

            rlo = jnp.maximum(sread(fr_v, t) - r0, 0)
            rhi = jnp.minimum(sread(fr_v, t + 1) - r0, rpw - 1)

            def row_body(r, c2):
                jlo = jnp.maximum(sread(rp_v, r), base)
                jhi = jnp.minimum(sread(rp_v, r + 1), base + K)

                def edge_body(j, c3):
                    jj = j - base
                    for g in range(ng):
                        sl = pl.ds(g * LANES, LANES)
                        plsc.addupdate(out_v.at[r, sl], rows_v[jj, sl])
                    return c3

                return lax.fori_loop(jlo, jhi, edge_body, c2)

            return lax.fori_loop(rlo, rhi + 1, row_body, c)

        lax.fori_loop(t0, t1, chunk_body, 0)

        def scale_body(r, c):
            dscale = sread(deg_v, r)
            for g in range(ng):
                sl = pl.ds(g * LANES, LANES)
                out_v[r, sl] = out_v[r, sl] * dscale
            return c

        lax.fori_loop(0, rpw, scale_body, 0)

        pltpu.sync_copy(out_v, out_hbm.at[pl.ds(r0, rpw)])

    return agg(Y, rp_pad, col_pad, deg_pad, fr_pad)


def kernel(X, weights, row_pointers, column_index, degrees, partPtr, part2Node,
           threadPerBlock):
    n = X.shape[0]
    e = column_index.shape[0]

    rpw = (-(-n // NW) + 7) // 8 * 8
    n_pad = NW * rpw

    Y = _matmul_scale(X, weights, degrees)

    rp_pad = jnp.concatenate(
        [row_pointers,
         jnp.full((n_pad + 24 - (n + 1),), e, dtype=jnp.int32)])
    col_pad = jnp.concatenate(
        [column_index, jnp.zeros((K,), dtype=jnp.int32)])
    deg_pad = jnp.concatenate(
        [degrees, jnp.zeros((n_pad + 16 - n,), dtype=jnp.float32)])

    nt = e // K
    grid = jnp.arange(0, nt + 1, dtype=jnp.int32) * K
    fr = (jnp.searchsorted(row_pointers, grid, side="right") - 1).astype(
        jnp.int32)
    nfr = (nt + 1 + LANES + 7) // 8 * 8
    fr_pad = jnp.concatenate(
        [fr, jnp.full((nfr - (nt + 1),), n - 1, dtype=jnp.int32)])

    out = _sc_aggregate(Y, rp_pad, col_pad, deg_pad, fr_pad, rpw)
    return out[:n]


# speedup vs baseline: 11.6354x; 1.2643x over previous
import functools

import jax
import jax.numpy as jnp
from jax import lax
from jax.experimental import pallas as pl
from jax.experimental.pallas import tpu as pltpu
from jax.experimental.pallas import tpu_sc as plsc

NC = 2
NS = 16
NW = NC * NS
LANES = 16
K = 128


def _mm_body(x_ref, w_ref, d_ref, y_ref):
    y = jnp.dot(x_ref[...], w_ref[...], preferred_element_type=jnp.float32)
    y_ref[...] = y * d_ref[...]


def _matmul_scale(X, W, deg):
    n, d_in = X.shape
    d_out = W.shape[1]
    bm = 1000
    grid = (n // bm,)
    return pl.pallas_call(
        _mm_body,
        grid=grid,
        in_specs=[
            pl.BlockSpec((bm, d_in), lambda i: (i, 0)),
            pl.BlockSpec((d_in, d_out), lambda i: (0, 0)),
            pl.BlockSpec((bm, 1), lambda i: (i, 0)),
        ],
        out_specs=pl.BlockSpec((bm, d_out), lambda i: (i, 0)),
        out_shape=jax.ShapeDtypeStruct((n, d_out), jnp.float32),
    )(X, W, deg[:, None])


G = 128


def _sc_aggregate(Y, rp_pad, col_pad, deg_pad, fr_pad, rpw):
    n_pad = NW * rpw
    d = Y.shape[1]
    ng = d // LANES
    nfr = fr_pad.shape[0]

    mesh = plsc.VectorSubcoreMesh(core_axis_name="c", subcore_axis_name="s")

    @functools.partial(
        pl.kernel,
        out_type=jax.ShapeDtypeStruct((n_pad, d), jnp.float32),
        mesh=mesh,
        scratch_types=[
            pltpu.VMEM((rpw + 24,), jnp.int32),
            pltpu.VMEM((rpw + 16,), jnp.float32),
            pltpu.VMEM((nfr,), jnp.int32),
            pltpu.VMEM((G * K,), jnp.int32),
            pltpu.VMEM((K, d), jnp.float32),
            pltpu.VMEM((K, d), jnp.float32),
            pltpu.VMEM((rpw, d), jnp.float32),
            pltpu.SemaphoreType.DMA,
            pltpu.SemaphoreType.DMA,
        ],
    )
    def agg(y_hbm, rp_hbm, colx_hbm, deg_hbm, fr_hbm, out_hbm,
            rp_v, deg_v, fr_v, col_v, rows0_v, rows1_v, out_v, sem0, sem1):
        wid = lax.axis_index("s") * NC + lax.axis_index("c")
        r0 = wid * rpw
        pltpu.sync_copy(rp_hbm.at[pl.ds(r0, rpw + 24)], rp_v)
        pltpu.sync_copy(deg_hbm.at[pl.ds(r0, rpw + 16)], deg_v)
        pltpu.sync_copy(fr_hbm, fr_v)

        def sread(ref, i):
            return ref[pl.ds(i, LANES)][0]

        zeros = jnp.zeros((LANES,), jnp.float32)

        def zero_body(r, c):
            for g in range(ng):
                out_v[r, pl.ds(g * LANES, LANES)] = zeros
            return c

        lax.fori_loop(0, rpw, zero_body, 0)

        e0 = sread(rp_v, 0)
        e1 = sread(rp_v, rpw)
        t0 = e0 // K
        t1 = (e1 + (K - 1)) // K
        tcount = t1 - t0
        nsup = (tcount + (G - 1)) // G

        def gather(li, rows_v, sem):
            return pltpu.make_async_copy(
                y_hbm.at[col_v.at[pl.ds(li * K, K)]], rows_v, sem)

        def accumulate(t, li, rows_v, valid):
            base = t * K
            off = li * K
            rlo = jnp.maximum(sread(fr_v, t) - r0, 0)
            rhi = jnp.minimum(sread(fr_v, t + 1) - r0, rpw - 1)
            rhi = jnp.where(valid, rhi, rlo - 1)

            def row_body(r, c2):
                jlo = jnp.maximum(sread(rp_v, r), base)
                jhi = jnp.minimum(sread(rp_v, r + 1), base + K)

                def edge_body(j, c3):
                    jj = j - base
                    for g in range(ng):
                        sl = pl.ds(g * LANES, LANES)
                        plsc.addupdate(out_v.at[r, sl], rows_v[jj, sl])
                    return c3

                return lax.fori_loop(jlo, jhi, edge_body, c2)

            lax.fori_loop(rlo, rhi + 1, row_body, 0)

        def super_body(s, c):
            ts = t0 + s * G
            nin = jnp.minimum(t1 - ts, G)
            pltpu.sync_copy(colx_hbm.at[pl.ds(ts * K, G * K)], col_v)

            gather(0, rows0_v, sem0).start()

            def pair_body(p, c2):
                u = 2 * p
                v = u + 1
                vc = jnp.minimum(v, nin - 1)
                gather(vc, rows1_v, sem1).start()
                gather(0, rows0_v, sem0).wait()
                accumulate(ts + u, u, rows0_v, u < nin)
                uc = jnp.minimum(u + 2, nin - 1)
                gather(uc, rows0_v, sem0).start()
                gather(0, rows1_v, sem1).wait()
                accumulate(ts + vc, vc, rows1_v, v < nin)
                return c2

            npairs = (nin + 1) // 2
            lax.fori_loop(0, npairs, pair_body, 0)
            gather(0, rows0_v, sem0).wait()
            return c

        lax.fori_loop(0, nsup, super_body, 0)

        def scale_body(r, c):
            dscale = sread(deg_v, r)
            for g in range(ng):
                sl = pl.ds(g * LANES, LANES)
                out_v[r, sl] = out_v[r, sl] * dscale
            return c

        lax.fori_loop(0, rpw, scale_body, 0)

        pltpu.sync_copy(out_v, out_hbm.at[pl.ds(r0, rpw)])

    return agg(Y, rp_pad, col_pad, deg_pad, fr_pad)


def kernel(X, weights, row_pointers, column_index, degrees, partPtr, part2Node,
           threadPerBlock):
    n = X.shape[0]
    e = column_index.shape[0]

    rpw = (-(-n // NW) + 7) // 8 * 8
    n_pad = NW * rpw

    Y = _matmul_scale(X, weights, degrees)

    rp_pad = jnp.concatenate(
        [row_pointers,
         jnp.full((n_pad + 24 - (n + 1),), e, dtype=jnp.int32)])
    col_pad = jnp.concatenate(
        [column_index, jnp.zeros((G * K,), dtype=jnp.int32)])
    deg_pad = jnp.concatenate(
        [degrees, jnp.zeros((n_pad + 16 - n,), dtype=jnp.float32)])

    nt = e // K
    grid = jnp.arange(0, nt + 1, dtype=jnp.int32) * K
    fr = (jnp.searchsorted(row_pointers, grid, side="right") - 1).astype(
        jnp.int32)
    nfr = (nt + 1 + LANES + 7) // 8 * 8
    fr_pad = jnp.concatenate(
        [fr, jnp.full((nfr - (nt + 1),), n - 1, dtype=jnp.int32)])

    out = _sc_aggregate(Y, rp_pad, col_pad, deg_pad, fr_pad, rpw)
    return out[:n]
